# initial kernel scaffold (unmeasured)
import jax
import jax.numpy as jnp
from jax import lax
from jax.experimental import pallas as pl
from jax.experimental.pallas import tpu as pltpu

N_DEV = 8


def kernel(x, w_mat):
    k_full, kblk = x.shape
    _, n_out = w_mat.shape
    m_per = k_full // N_DEV

    def body(x_ref, w_ref, out_ref, xrow_ref, amax_ref,
             send_sems, recv_sems, amax_send_sems, amax_recv_sems):
        my = lax.axis_index("i")

        barrier = pltpu.get_barrier_semaphore()
        for o in range(1, N_DEV):
            other = lax.rem(my + o, N_DEV)
            pl.semaphore_signal(barrier, inc=1, device_id=(other,),
                                device_id_type=pl.DeviceIdType.MESH)
        pl.semaphore_wait(barrier, N_DEV - 1)

        sends = []
        for o in range(1, N_DEV):
            tgt = lax.rem(my + o, N_DEV)
            rdma = pltpu.make_async_remote_copy(
                src_ref=x_ref.at[pl.ds(tgt * m_per, m_per), :],
                dst_ref=xrow_ref.at[:, pl.ds(my * kblk, kblk)],
                send_sem=send_sems.at[tgt],
                recv_sem=recv_sems.at[my],
                device_id=(tgt,),
                device_id_type=pl.DeviceIdType.MESH,
            )
            rdma.start()
            sends.append(rdma)

        xrow_ref[:, pl.ds(my * kblk, kblk)] = x_ref[pl.ds(my * m_per, m_per), :]

        for o in range(1, N_DEV):
            origin = lax.rem(my + N_DEV - o, N_DEV)
            recv = pltpu.make_async_remote_copy(
                src_ref=x_ref.at[pl.ds(0, m_per), :],
                dst_ref=xrow_ref.at[:, pl.ds(origin * kblk, kblk)],
                send_sem=send_sems.at[origin],
                recv_sem=recv_sems.at[origin],
                device_id=(origin,),
                device_id_type=pl.DeviceIdType.MESH,
            )
            recv.wait_recv()

        y = jnp.dot(xrow_ref[:, :], w_ref[:, :],
                    preferred_element_type=jnp.float32)

        local_amax = jnp.max(jnp.abs(y))
        amax_ref[pl.ds(my, 1), :] = jnp.full((1, 128), local_amax,
                                             dtype=jnp.float32)
        amax_sends = []
        for o in range(1, N_DEV):
            tgt = lax.rem(my + o, N_DEV)
            r = pltpu.make_async_remote_copy(
                src_ref=amax_ref.at[pl.ds(my, 1), :],
                dst_ref=amax_ref.at[pl.ds(my, 1), :],
                send_sem=amax_send_sems.at[tgt],
                recv_sem=amax_recv_sems.at[my],
                device_id=(tgt,),
                device_id_type=pl.DeviceIdType.MESH,
            )
            r.start()
            amax_sends.append(r)
        for o in range(1, N_DEV):
            origin = lax.rem(my + N_DEV - o, N_DEV)
            rr = pltpu.make_async_remote_copy(
                src_ref=amax_ref.at[pl.ds(origin, 1), :],
                dst_ref=amax_ref.at[pl.ds(origin, 1), :],
                send_sem=amax_send_sems.at[origin],
                recv_sem=amax_recv_sems.at[origin],
                device_id=(origin,),
                device_id_type=pl.DeviceIdType.MESH,
            )
            rr.wait_recv()

        gmax = jnp.max(amax_ref[:, :])
        scale = gmax / 448.0
        q = jnp.clip(y / scale, -448.0, 448.0).astype(jnp.float8_e4m3fn)
        out_ref[:, :] = q.astype(jnp.float32) * scale

        for r in sends:
            r.wait_send()
        for r in amax_sends:
            r.wait_send()

    return pl.pallas_call(
        body,
        out_shape=jax.ShapeDtypeStruct((m_per, n_out), jnp.float32),
        in_specs=[
            pl.BlockSpec(memory_space=pltpu.VMEM),
            pl.BlockSpec(memory_space=pltpu.VMEM),
        ],
        out_specs=pl.BlockSpec(memory_space=pltpu.VMEM),
        scratch_shapes=[
            pltpu.VMEM((m_per, k_full), x.dtype),
            pltpu.VMEM((N_DEV, 128), jnp.float32),
            pltpu.SemaphoreType.DMA((N_DEV,)),
            pltpu.SemaphoreType.DMA((N_DEV,)),
            pltpu.SemaphoreType.DMA((N_DEV,)),
            pltpu.SemaphoreType.DMA((N_DEV,)),
        ],
        compiler_params=pltpu.CompilerParams(collective_id=0),
    )(x, w_mat)


# baseline (device time: 62257 ns/iter reference)
import jax
import jax.numpy as jnp
from jax import lax
from jax.experimental import pallas as pl
from jax.experimental.pallas import tpu as pltpu

N_DEV = 8


def kernel(x, w_mat):
    k_full, kblk = x.shape
    _, n_out = w_mat.shape
    m_per = k_full // N_DEV

    def body(x_ref, w_ref, out_ref, xbf_ref, xrow_ref, wbuf_ref, wbf_ref,
             amax_ref, wsems, send_sems, recv_sems, amax_send_sems,
             amax_recv_sems):
        my = lax.axis_index("i")

        barrier = pltpu.get_barrier_semaphore()
        for o in range(1, N_DEV):
            other = lax.rem(my + o, N_DEV)
            pl.semaphore_signal(barrier, inc=1, device_id=(other,),
                                device_id_type=pl.DeviceIdType.MESH)
        pl.semaphore_wait(barrier, N_DEV - 1)

        def w_copy(j, slot):
            return pltpu.make_async_copy(
                w_ref.at[pl.ds(j * m_per, m_per), :],
                wbuf_ref.at[slot],
                wsems.at[slot],
            )

        w_copy(0, 0).start()
        w_copy(1, 1).start()

        xbf_ref[:, :] = x_ref[:, :].astype(jnp.bfloat16)

        sends = []
        for o in range(1, N_DEV):
            tgt = lax.rem(my + o, N_DEV)
            rdma = pltpu.make_async_remote_copy(
                src_ref=xbf_ref.at[pl.ds(tgt * m_per, m_per), :],
                dst_ref=xrow_ref.at[:, pl.ds(my * kblk, kblk)],
                send_sem=send_sems.at[tgt],
                recv_sem=recv_sems.at[my],
                device_id=(tgt,),
                device_id_type=pl.DeviceIdType.MESH,
            )
            rdma.start()
            sends.append(rdma)

        xrow_ref[:, pl.ds(my * kblk, kblk)] = xbf_ref[pl.ds(my * m_per, m_per), :]

        for j in range(N_DEV):
            @pl.when(j != my)
            def _():
                pltpu.make_async_remote_copy(
                    src_ref=xbf_ref.at[pl.ds(0, m_per), :],
                    dst_ref=xrow_ref.at[:, pl.ds(j * kblk, kblk)],
                    send_sem=send_sems.at[j],
                    recv_sem=recv_sems.at[j],
                    device_id=(j,),
                    device_id_type=pl.DeviceIdType.MESH,
                ).wait_recv()

            slot = j % 2
            w_copy(j, slot).wait()
            wbf_ref[slot] = wbuf_ref[slot].astype(jnp.bfloat16)
            if j + 2 < N_DEV:
                w_copy(j + 2, slot).start()
            partial = jnp.dot(xrow_ref[:, j * kblk:(j + 1) * kblk],
                              wbf_ref[slot],
                              preferred_element_type=jnp.float32)
            if j == 0:
                out_ref[:, :] = partial
            else:
                out_ref[:, :] = out_ref[:, :] + partial

        y = out_ref[:, :]
        local_amax = jnp.max(jnp.abs(y))
        amax_ref[pl.ds(my, 1), :] = jnp.full((1, 128), local_amax,
                                             dtype=jnp.float32)
        amax_sends = []
        for o in range(1, N_DEV):
            tgt = lax.rem(my + o, N_DEV)
            r = pltpu.make_async_remote_copy(
                src_ref=amax_ref.at[pl.ds(my, 1), :],
                dst_ref=amax_ref.at[pl.ds(my, 1), :],
                send_sem=amax_send_sems.at[tgt],
                recv_sem=amax_recv_sems.at[my],
                device_id=(tgt,),
                device_id_type=pl.DeviceIdType.MESH,
            )
            r.start()
            amax_sends.append(r)
        for o in range(1, N_DEV):
            origin = lax.rem(my + N_DEV - o, N_DEV)
            pltpu.make_async_remote_copy(
                src_ref=amax_ref.at[pl.ds(origin, 1), :],
                dst_ref=amax_ref.at[pl.ds(origin, 1), :],
                send_sem=amax_send_sems.at[origin],
                recv_sem=amax_recv_sems.at[origin],
                device_id=(origin,),
                device_id_type=pl.DeviceIdType.MESH,
            ).wait_recv()

        gmax = jnp.max(amax_ref[:, :])
        scale = gmax / 448.0
        q = jnp.clip(y / scale, -448.0, 448.0).astype(jnp.float8_e4m3fn)
        out_ref[:, :] = q.astype(jnp.float32) * scale

        for r in sends:
            r.wait_send()
        for r in amax_sends:
            r.wait_send()

    return pl.pallas_call(
        body,
        out_shape=jax.ShapeDtypeStruct((m_per, n_out), jnp.float32),
        in_specs=[
            pl.BlockSpec(memory_space=pltpu.VMEM),
            pl.BlockSpec(memory_space=pl.ANY),
        ],
        out_specs=pl.BlockSpec(memory_space=pltpu.VMEM),
        scratch_shapes=[
            pltpu.VMEM((k_full, kblk), jnp.bfloat16),
            pltpu.VMEM((m_per, k_full), jnp.bfloat16),
            pltpu.VMEM((2, m_per, n_out), jnp.float32),
            pltpu.VMEM((2, m_per, n_out), jnp.bfloat16),
            pltpu.VMEM((N_DEV, 128), jnp.float32),
            pltpu.SemaphoreType.DMA((2,)),
            pltpu.SemaphoreType.DMA((N_DEV,)),
            pltpu.SemaphoreType.DMA((N_DEV,)),
            pltpu.SemaphoreType.DMA((N_DEV,)),
            pltpu.SemaphoreType.DMA((N_DEV,)),
        ],
        compiler_params=pltpu.CompilerParams(
            collective_id=0,
            vmem_limit_bytes=64 * 1024 * 1024,
        ),
    )(x, w_mat)


# device time: 59684 ns/iter; 1.0431x vs baseline; 1.0431x over previous
import jax
import jax.numpy as jnp
from jax import lax
from jax.experimental import pallas as pl
from jax.experimental.pallas import tpu as pltpu

N_DEV = 8


def kernel(x, w_mat):
    k_full, kblk = x.shape
    _, n_out = w_mat.shape
    m_per = k_full // N_DEV

    def body(x_ref, w_ref, out_ref, xbf_ref, xrow_ref, wbuf_ref,
             amax_ref, wsems, send_sems, recv_sems, amax_send_sems,
             amax_recv_sems):
        my = lax.axis_index("i")

        barrier = pltpu.get_barrier_semaphore()
        for o in range(1, N_DEV):
            other = lax.rem(my + o, N_DEV)
            pl.semaphore_signal(barrier, inc=1, device_id=(other,),
                                device_id_type=pl.DeviceIdType.MESH)
        pl.semaphore_wait(barrier, N_DEV - 1)

        def w_copy(j, slot):
            return pltpu.make_async_copy(
                w_ref.at[pl.ds(j * m_per, m_per), :],
                wbuf_ref.at[slot],
                wsems.at[slot],
            )

        w_copy(0, 0).start()
        w_copy(1, 1).start()
        w_copy(2, 2).start()

        xbf_ref[:, :] = x_ref[:, :].astype(jnp.bfloat16)

        sends = []
        for o in range(1, N_DEV):
            tgt = lax.rem(my + o, N_DEV)
            rdma = pltpu.make_async_remote_copy(
                src_ref=xbf_ref.at[pl.ds(tgt * m_per, m_per), :],
                dst_ref=xrow_ref.at[:, pl.ds(my * kblk, kblk)],
                send_sem=send_sems.at[tgt],
                recv_sem=recv_sems.at[my],
                device_id=(tgt,),
                device_id_type=pl.DeviceIdType.MESH,
            )
            rdma.start()
            sends.append(rdma)

        xrow_ref[:, pl.ds(my * kblk, kblk)] = xbf_ref[pl.ds(my * m_per, m_per), :]

        for j in range(N_DEV):
            @pl.when(j != my)
            def _():
                pltpu.make_async_remote_copy(
                    src_ref=xbf_ref.at[pl.ds(0, m_per), :],
                    dst_ref=xrow_ref.at[:, pl.ds(j * kblk, kblk)],
                    send_sem=send_sems.at[j],
                    recv_sem=recv_sems.at[j],
                    device_id=(j,),
                    device_id_type=pl.DeviceIdType.MESH,
                ).wait_recv()

            slot = j % 3
            w_copy(j, slot).wait()
            if j + 3 < N_DEV:
                w_copy(j + 3, slot).start()
            partial = lax.dot_general(
                xrow_ref[:, j * kblk:(j + 1) * kblk].astype(jnp.float32),
                wbuf_ref[slot],
                (((1,), (0,)), ((), ())),
                precision=lax.Precision.DEFAULT,
                preferred_element_type=jnp.float32,
            )
            if j == 0:
                out_ref[:, :] = partial
            else:
                out_ref[:, :] = out_ref[:, :] + partial

        y = out_ref[:, :]
        local_amax = jnp.max(jnp.abs(y))
        amax_ref[pl.ds(my, 1), :] = jnp.full((1, 128), local_amax,
                                             dtype=jnp.float32)
        amax_sends = []
        for o in range(1, N_DEV):
            tgt = lax.rem(my + o, N_DEV)
            r = pltpu.make_async_remote_copy(
                src_ref=amax_ref.at[pl.ds(my, 1), :],
                dst_ref=amax_ref.at[pl.ds(my, 1), :],
                send_sem=amax_send_sems.at[tgt],
                recv_sem=amax_recv_sems.at[my],
                device_id=(tgt,),
                device_id_type=pl.DeviceIdType.MESH,
            )
            r.start()
            amax_sends.append(r)
        for o in range(1, N_DEV):
            origin = lax.rem(my + N_DEV - o, N_DEV)
            pltpu.make_async_remote_copy(
                src_ref=amax_ref.at[pl.ds(origin, 1), :],
                dst_ref=amax_ref.at[pl.ds(origin, 1), :],
                send_sem=amax_send_sems.at[origin],
                recv_sem=amax_recv_sems.at[origin],
                device_id=(origin,),
                device_id_type=pl.DeviceIdType.MESH,
            ).wait_recv()

        gmax = jnp.max(amax_ref[:, :])
        scale = gmax / 448.0
        q = jnp.clip(y / scale, -448.0, 448.0).astype(jnp.float8_e4m3fn)
        out_ref[:, :] = q.astype(jnp.float32) * scale

        for r in sends:
            r.wait_send()
        for r in amax_sends:
            r.wait_send()

    return pl.pallas_call(
        body,
        out_shape=jax.ShapeDtypeStruct((m_per, n_out), jnp.float32),
        in_specs=[
            pl.BlockSpec(memory_space=pltpu.VMEM),
            pl.BlockSpec(memory_space=pl.ANY),
        ],
        out_specs=pl.BlockSpec(memory_space=pltpu.VMEM),
        scratch_shapes=[
            pltpu.VMEM((k_full, kblk), jnp.bfloat16),
            pltpu.VMEM((m_per, k_full), jnp.bfloat16),
            pltpu.VMEM((3, m_per, n_out), jnp.float32),
            pltpu.VMEM((N_DEV, 128), jnp.float32),
            pltpu.SemaphoreType.DMA((3,)),
            pltpu.SemaphoreType.DMA((N_DEV,)),
            pltpu.SemaphoreType.DMA((N_DEV,)),
            pltpu.SemaphoreType.DMA((N_DEV,)),
            pltpu.SemaphoreType.DMA((N_DEV,)),
        ],
        compiler_params=pltpu.CompilerParams(
            collective_id=0,
            vmem_limit_bytes=64 * 1024 * 1024,
        ),
    )(x, w_mat)


# device time: 59664 ns/iter; 1.0435x vs baseline; 1.0003x over previous
import jax
import jax.numpy as jnp
from jax import lax
from jax.experimental import pallas as pl
from jax.experimental.pallas import tpu as pltpu

N_DEV = 8


def kernel(x, w_mat):
    k_full, kblk = x.shape
    _, n_out = w_mat.shape
    m_per = k_full // N_DEV

    def body(x_ref, w_ref, out_ref, xbf_ref, xrow_ref, wbuf_ref,
             amax_ref, wsems, send_sems, recv_sems, amax_send_sems,
             amax_recv_sems):
        my = lax.axis_index("i")

        barrier = pltpu.get_barrier_semaphore()
        for o in range(1, N_DEV):
            other = lax.rem(my + o, N_DEV)
            pl.semaphore_signal(barrier, inc=1, device_id=(other,),
                                device_id_type=pl.DeviceIdType.MESH)
        pl.semaphore_wait(barrier, N_DEV - 1)

        def w_copy(j, slot):
            return pltpu.make_async_copy(
                w_ref.at[pl.ds(j * m_per, m_per), :],
                wbuf_ref.at[slot],
                wsems.at[slot],
            )

        w_copy(0, 0).start()
        w_copy(1, 1).start()
        w_copy(2, 2).start()

        xbf_ref[:, :] = x_ref[:, :].astype(jnp.bfloat16)

        sends = []
        for o in range(1, N_DEV):
            tgt = lax.rem(my + o, N_DEV)
            rdma = pltpu.make_async_remote_copy(
                src_ref=xbf_ref.at[pl.ds(tgt * m_per, m_per), :],
                dst_ref=xrow_ref.at[:, pl.ds(my * kblk, kblk)],
                send_sem=send_sems.at[tgt],
                recv_sem=recv_sems.at[my],
                device_id=(tgt,),
                device_id_type=pl.DeviceIdType.MESH,
            )
            rdma.start()
            sends.append(rdma)

        xrow_ref[:, pl.ds(my * kblk, kblk)] = xbf_ref[pl.ds(my * m_per, m_per), :]

        for j in range(N_DEV):
            @pl.when(j != my)
            def _():
                pltpu.make_async_remote_copy(
                    src_ref=xbf_ref.at[pl.ds(0, m_per), :],
                    dst_ref=xrow_ref.at[:, pl.ds(j * kblk, kblk)],
                    send_sem=send_sems.at[j],
                    recv_sem=recv_sems.at[j],
                    device_id=(j,),
                    device_id_type=pl.DeviceIdType.MESH,
                ).wait_recv()

            slot = j % 3
            w_copy(j, slot).wait()
            partial = lax.dot_general(
                xrow_ref[:, j * kblk:(j + 1) * kblk].astype(jnp.float32),
                wbuf_ref[slot],
                (((1,), (0,)), ((), ())),
                precision=lax.Precision.DEFAULT,
                preferred_element_type=jnp.float32,
            )
            if j + 3 < N_DEV:
                w_copy(j + 3, slot).start()
            if j == 0:
                out_ref[:, :] = partial
            else:
                out_ref[:, :] = out_ref[:, :] + partial

        y = out_ref[:, :]
        local_amax = jnp.max(jnp.abs(y))
        amax_ref[pl.ds(my, 1), :] = jnp.full((1, 128), local_amax,
                                             dtype=jnp.float32)
        amax_sends = []
        for o in range(1, N_DEV):
            tgt = lax.rem(my + o, N_DEV)
            r = pltpu.make_async_remote_copy(
                src_ref=amax_ref.at[pl.ds(my, 1), :],
                dst_ref=amax_ref.at[pl.ds(my, 1), :],
                send_sem=amax_send_sems.at[tgt],
                recv_sem=amax_recv_sems.at[my],
                device_id=(tgt,),
                device_id_type=pl.DeviceIdType.MESH,
            )
            r.start()
            amax_sends.append(r)
        for o in range(1, N_DEV):
            origin = lax.rem(my + N_DEV - o, N_DEV)
            pltpu.make_async_remote_copy(
                src_ref=amax_ref.at[pl.ds(origin, 1), :],
                dst_ref=amax_ref.at[pl.ds(origin, 1), :],
                send_sem=amax_send_sems.at[origin],
                recv_sem=amax_recv_sems.at[origin],
                device_id=(origin,),
                device_id_type=pl.DeviceIdType.MESH,
            ).wait_recv()

        gmax = jnp.max(amax_ref[:, :])
        scale = gmax / 448.0
        q = jnp.clip(y / scale, -448.0, 448.0).astype(jnp.float8_e4m3fn)
        out_ref[:, :] = q.astype(jnp.float32) * scale

        for r in sends:
            r.wait_send()
        for r in amax_sends:
            r.wait_send()

    return pl.pallas_call(
        body,
        out_shape=jax.ShapeDtypeStruct((m_per, n_out), jnp.float32),
        in_specs=[
            pl.BlockSpec(memory_space=pltpu.VMEM),
            pl.BlockSpec(memory_space=pl.ANY),
        ],
        out_specs=pl.BlockSpec(memory_space=pltpu.VMEM),
        scratch_shapes=[
            pltpu.VMEM((k_full, kblk), jnp.bfloat16),
            pltpu.VMEM((m_per, k_full), jnp.bfloat16),
            pltpu.VMEM((3, m_per, n_out), jnp.float32),
            pltpu.VMEM((N_DEV, 128), jnp.float32),
            pltpu.SemaphoreType.DMA((3,)),
            pltpu.SemaphoreType.DMA((N_DEV,)),
            pltpu.SemaphoreType.DMA((N_DEV,)),
            pltpu.SemaphoreType.DMA((N_DEV,)),
            pltpu.SemaphoreType.DMA((N_DEV,)),
        ],
        compiler_params=pltpu.CompilerParams(
            collective_id=0,
            vmem_limit_bytes=64 * 1024 * 1024,
        ),
    )(x, w_mat)
